# baseline (speedup 1.0000x reference)
import functools

import jax
import jax.numpy as jnp
from jax import lax
from jax.experimental import pallas as pl
from jax.experimental.pallas import tpu as pltpu
from jax.experimental.pallas import tpu_sc as plsc

_N = 10000
_E = 320000
_H = 128
_G = 64
_C = 10

_NC = 2
_NS = 16
_NW = _NC * _NS
_EPT = _E // _NW
_EBATCH = 125
_NCH = _EPT // _EBATCH
_HCH = _NCH // 2
_ZR = _N // _NS
_FR = 632
_FL = _N - (_NS - 1) * _FR

_mesh = plsc.VectorSubcoreMesh(
    core_axis_name="c", subcore_axis_name="s", num_cores=_NC, num_subcores=_NS
)


@functools.partial(
    pl.kernel,
    out_type=jax.ShapeDtypeStruct((_NC, _N, _H), jnp.float32),
    mesh=_mesh,
    scratch_types=[
        pltpu.VMEM((2 * _HCH, _EBATCH), jnp.int32),
        pltpu.VMEM((2, _EBATCH, _H), jnp.float32),
        pltpu.VMEM_SHARED((_N, _H), jnp.float32),
        pltpu.SemaphoreType.DMA((2,)),
    ],
)
def _seg_sum(h, src, dst, zeros, out, idx_v, rows_v, acc, sem):
    c = lax.axis_index("c")
    s = lax.axis_index("s")
    wid = c * _NS + s
    pltpu.sync_copy(zeros, acc.at[pl.ds(s * _ZR, _ZR)])
    plsc.subcore_barrier()

    for half in range(2):
        pltpu.sync_copy(src.at[half].at[wid], idx_v.at[pl.ds(0, _HCH)])
        pltpu.sync_copy(dst.at[half].at[wid], idx_v.at[pl.ds(_HCH, _HCH)])
        pltpu.async_copy(h.at[idx_v.at[0]], rows_v.at[0], sem.at[0])

        def body(j, carry):
            nxt = j + 1

            @pl.when(nxt < _HCH)
            def _():
                nb = lax.rem(nxt, 2)
                pltpu.async_copy(h.at[idx_v.at[nxt]], rows_v.at[nb], sem.at[nb])

            jb = lax.rem(j, 2)
            pltpu.make_async_copy(
                h.at[idx_v.at[j]], rows_v.at[jb], sem.at[jb]
            ).wait()
            pltpu.sync_copy(rows_v.at[jb], acc.at[idx_v.at[_HCH + j]], add=True)
            return carry

        lax.fori_loop(0, _HCH, body, 0)
    plsc.subcore_barrier()

    @pl.when(s < _NS - 1)
    def _():
        pltpu.sync_copy(
            acc.at[pl.ds(s * _FR, _FR)], out.at[c].at[pl.ds(s * _FR, _FR)]
        )

    @pl.when(s == _NS - 1)
    def _():
        pltpu.sync_copy(
            acc.at[pl.ds((_NS - 1) * _FR, _FL)],
            out.at[c].at[pl.ds((_NS - 1) * _FR, _FL)],
        )


_BR = 1000


def _self_body(x, ws, b, o):
    o[...] = (
        jnp.dot(x[...], ws[...], preferred_element_type=jnp.float32) + b[...]
    )


def _self_linear(x, ws, b):
    return pl.pallas_call(
        _self_body,
        grid=(_N // _BR,),
        in_specs=[
            pl.BlockSpec((_BR, _H), lambda i: (i, 0)),
            pl.BlockSpec((_H, _H), lambda i: (0, 0)),
            pl.BlockSpec((1, _H), lambda i: (0, 0)),
        ],
        out_specs=pl.BlockSpec((_BR, _H), lambda i: (i, 0)),
        out_shape=jax.ShapeDtypeStruct((_N, _H), jnp.float32),
    )(x, ws, b)


def _comb_body(a0, a1, sf, wr, o):
    y = jnp.dot(a0[...] + a1[...], wr[...], preferred_element_type=jnp.float32)
    o[...] = jnp.maximum(y + sf[...], 0.0)


def _combine_relu(a0, a1, sf, wr):
    return pl.pallas_call(
        _comb_body,
        grid=(_N // _BR,),
        in_specs=[
            pl.BlockSpec((_BR, _H), lambda i: (i, 0)),
            pl.BlockSpec((_BR, _H), lambda i: (i, 0)),
            pl.BlockSpec((_BR, _H), lambda i: (i, 0)),
            pl.BlockSpec((_H, _H), lambda i: (0, 0)),
        ],
        out_specs=pl.BlockSpec((_BR, _H), lambda i: (i, 0)),
        out_shape=jax.ShapeDtypeStruct((_N, _H), jnp.float32),
    )(a0, a1, sf, wr)


def _last_body(a0, a1, sf, wr, batch, wlin, blin, o, sums, counts):
    i = pl.program_id(0)

    @pl.when(i == 0)
    def _():
        sums[...] = jnp.zeros_like(sums)
        counts[...] = jnp.zeros_like(counts)

    y = jnp.dot(a0[...] + a1[...], wr[...], preferred_element_type=jnp.float32)
    y = y + sf[...]
    oh = (lax.broadcasted_iota(jnp.int32, (_BR, _G), 1) == batch[...]).astype(
        jnp.float32
    )
    sums[...] += lax.dot_general(
        oh, y, (((0,), (0,)), ((), ())), preferred_element_type=jnp.float32
    )
    counts[...] += lax.dot_general(
        oh,
        jnp.ones((_BR, 1), jnp.float32),
        (((0,), (0,)), ((), ())),
        preferred_element_type=jnp.float32,
    )

    @pl.when(i == pl.num_programs(0) - 1)
    def _():
        pooled = sums[...] / jnp.maximum(counts[...], 1.0)
        o[...] = (
            jnp.dot(pooled, wlin[...], preferred_element_type=jnp.float32) + blin[...]
        )


def _last_layer_pool(a0, a1, sf, wr, batch2d, wlin, blin):
    return pl.pallas_call(
        _last_body,
        grid=(_N // _BR,),
        in_specs=[
            pl.BlockSpec((_BR, _H), lambda i: (i, 0)),
            pl.BlockSpec((_BR, _H), lambda i: (i, 0)),
            pl.BlockSpec((_BR, _H), lambda i: (i, 0)),
            pl.BlockSpec((_H, _H), lambda i: (0, 0)),
            pl.BlockSpec((_BR, 1), lambda i: (i, 0)),
            pl.BlockSpec((_H, _C), lambda i: (0, 0)),
            pl.BlockSpec((1, _C), lambda i: (0, 0)),
        ],
        out_specs=pl.BlockSpec((_G, _C), lambda i: (0, 0)),
        out_shape=jax.ShapeDtypeStruct((_G, _C), jnp.float32),
        scratch_shapes=[
            pltpu.VMEM((_G, _H), jnp.float32),
            pltpu.VMEM((_G, 1), jnp.float32),
        ],
    )(a0, a1, sf, wr, batch2d, wlin, blin)


def kernel(x, edge_index, batch, W1r, W1s, b1, W2r, W2s, b2, W3r, W3s, b3, Wlin, blin):
    src4 = edge_index[0].reshape(2, _NW, _HCH, _EBATCH)
    dst4 = edge_index[1].reshape(2, _NW, _HCH, _EBATCH)
    zeros = jnp.zeros((_ZR, _H), jnp.float32)
    batch2d = batch.reshape(_N, 1).astype(jnp.int32)

    h = x
    for wr, ws, b in ((W1r, W1s, b1), (W2r, W2s, b2)):
        sf = _self_linear(h, ws, b.reshape(1, _H))
        accs = _seg_sum(h, src4, dst4, zeros)
        h = _combine_relu(accs[0], accs[1], sf, wr)
    sf = _self_linear(h, W3s, b3.reshape(1, _H))
    accs = _seg_sum(h, src4, dst4, zeros)
    return _last_layer_pool(
        accs[0], accs[1], sf, W3r, batch2d, Wlin, blin.reshape(1, _C)
    )


# speedup vs baseline: 1.0115x; 1.0115x over previous
import functools

import jax
import jax.numpy as jnp
from jax import lax
from jax.experimental import pallas as pl
from jax.experimental.pallas import tpu as pltpu
from jax.experimental.pallas import tpu_sc as plsc

_N = 10000
_E = 320000
_H = 128
_G = 64
_C = 10

_NC = 2
_NS = 16
_NW = _NC * _NS
_EPT = _E // _NW
_EBATCH = 125
_NCH = _EPT // _EBATCH
_HCH = _NCH // 2
_ZR = _N // _NS
_FR = 632
_FL = _N - (_NS - 1) * _FR

_mesh = plsc.VectorSubcoreMesh(
    core_axis_name="c", subcore_axis_name="s", num_cores=_NC, num_subcores=_NS
)


@functools.partial(
    pl.kernel,
    out_type=jax.ShapeDtypeStruct((_NC, _N, _H), jnp.float32),
    mesh=_mesh,
    scratch_types=[
        pltpu.VMEM((2 * _HCH, _EBATCH), jnp.int32),
        pltpu.VMEM((2, _EBATCH, _H), jnp.float32),
        pltpu.VMEM_SHARED((_N, _H), jnp.float32),
        pltpu.SemaphoreType.DMA((2,)),
    ],
)
def _seg_sum(h, src, dst, zeros, out, idx_v, rows_v, acc, sem):
    c = lax.axis_index("c")
    s = lax.axis_index("s")
    wid = c * _NS + s
    pltpu.sync_copy(zeros, acc.at[pl.ds(s * _ZR, _ZR)])
    plsc.subcore_barrier()

    for half in range(2):
        pltpu.sync_copy(src.at[half].at[wid], idx_v.at[pl.ds(0, _HCH)])
        pltpu.sync_copy(dst.at[half].at[wid], idx_v.at[pl.ds(_HCH, _HCH)])
        pltpu.async_copy(h.at[idx_v.at[0]], rows_v.at[0], sem.at[0])

        def body(j, carry):
            nxt = j + 1

            @pl.when(nxt < _HCH)
            def _():
                nb = lax.rem(nxt, 2)
                pltpu.async_copy(h.at[idx_v.at[nxt]], rows_v.at[nb], sem.at[nb])

            jb = lax.rem(j, 2)
            pltpu.make_async_copy(
                h.at[idx_v.at[j]], rows_v.at[jb], sem.at[jb]
            ).wait()
            pltpu.sync_copy(rows_v.at[jb], acc.at[idx_v.at[_HCH + j]], add=True)
            return carry

        lax.fori_loop(0, _HCH, body, 0)
    plsc.subcore_barrier()

    @pl.when(s < _NS - 1)
    def _():
        pltpu.sync_copy(
            acc.at[pl.ds(s * _FR, _FR)], out.at[c].at[pl.ds(s * _FR, _FR)]
        )

    @pl.when(s == _NS - 1)
    def _():
        pltpu.sync_copy(
            acc.at[pl.ds((_NS - 1) * _FR, _FL)],
            out.at[c].at[pl.ds((_NS - 1) * _FR, _FL)],
        )


_BR = 1000


def _lin_body(a0, a1, x, wr, ws, b, o):
    y = jnp.dot(a0[...] + a1[...], wr[...], preferred_element_type=jnp.float32)
    y = y + jnp.dot(x[...], ws[...], preferred_element_type=jnp.float32)
    o[...] = jnp.maximum(y + b[...], 0.0)


def _fused_linear_relu(a0, a1, x, wr, ws, b):
    return pl.pallas_call(
        _lin_body,
        grid=(_N // _BR,),
        in_specs=[
            pl.BlockSpec((_BR, _H), lambda i: (i, 0)),
            pl.BlockSpec((_BR, _H), lambda i: (i, 0)),
            pl.BlockSpec((_BR, _H), lambda i: (i, 0)),
            pl.BlockSpec((_H, _H), lambda i: (0, 0)),
            pl.BlockSpec((_H, _H), lambda i: (0, 0)),
            pl.BlockSpec((1, _H), lambda i: (0, 0)),
        ],
        out_specs=pl.BlockSpec((_BR, _H), lambda i: (i, 0)),
        out_shape=jax.ShapeDtypeStruct((_N, _H), jnp.float32),
    )(a0, a1, x, wr, ws, b)


def _last_body(a0, a1, x, wr, ws, b, batch, wlin, blin, o, sums, counts):
    i = pl.program_id(0)

    @pl.when(i == 0)
    def _():
        sums[...] = jnp.zeros_like(sums)
        counts[...] = jnp.zeros_like(counts)

    y = jnp.dot(a0[...] + a1[...], wr[...], preferred_element_type=jnp.float32)
    y = y + jnp.dot(x[...], ws[...], preferred_element_type=jnp.float32)
    y = y + b[...]
    oh = (lax.broadcasted_iota(jnp.int32, (_BR, _G), 1) == batch[...]).astype(
        jnp.float32
    )
    sums[...] += lax.dot_general(
        oh, y, (((0,), (0,)), ((), ())), preferred_element_type=jnp.float32
    )
    counts[...] += lax.dot_general(
        oh,
        jnp.ones((_BR, 1), jnp.float32),
        (((0,), (0,)), ((), ())),
        preferred_element_type=jnp.float32,
    )

    @pl.when(i == pl.num_programs(0) - 1)
    def _():
        pooled = sums[...] / jnp.maximum(counts[...], 1.0)
        o[...] = (
            jnp.dot(pooled, wlin[...], preferred_element_type=jnp.float32) + blin[...]
        )


def _last_layer_pool(a0, a1, x, wr, ws, b, batch2d, wlin, blin):
    return pl.pallas_call(
        _last_body,
        grid=(_N // _BR,),
        in_specs=[
            pl.BlockSpec((_BR, _H), lambda i: (i, 0)),
            pl.BlockSpec((_BR, _H), lambda i: (i, 0)),
            pl.BlockSpec((_BR, _H), lambda i: (i, 0)),
            pl.BlockSpec((_H, _H), lambda i: (0, 0)),
            pl.BlockSpec((_H, _H), lambda i: (0, 0)),
            pl.BlockSpec((1, _H), lambda i: (0, 0)),
            pl.BlockSpec((_BR, 1), lambda i: (i, 0)),
            pl.BlockSpec((_H, _C), lambda i: (0, 0)),
            pl.BlockSpec((1, _C), lambda i: (0, 0)),
        ],
        out_specs=pl.BlockSpec((_G, _C), lambda i: (0, 0)),
        out_shape=jax.ShapeDtypeStruct((_G, _C), jnp.float32),
        scratch_shapes=[
            pltpu.VMEM((_G, _H), jnp.float32),
            pltpu.VMEM((_G, 1), jnp.float32),
        ],
    )(a0, a1, x, wr, ws, b, batch2d, wlin, blin)


def kernel(x, edge_index, batch, W1r, W1s, b1, W2r, W2s, b2, W3r, W3s, b3, Wlin, blin):
    src4 = edge_index[0].reshape(2, _NW, _HCH, _EBATCH)
    dst4 = edge_index[1].reshape(2, _NW, _HCH, _EBATCH)
    zeros = jnp.zeros((_ZR, _H), jnp.float32)
    batch2d = batch.reshape(_N, 1).astype(jnp.int32)

    h = x
    for wr, ws, b in ((W1r, W1s, b1), (W2r, W2s, b2)):
        accs = _seg_sum(h, src4, dst4, zeros)
        h = _fused_linear_relu(accs[0], accs[1], h, wr, ws, b.reshape(1, _H))
    accs = _seg_sum(h, src4, dst4, zeros)
    return _last_layer_pool(
        accs[0], accs[1], h, W3r, W3s, b3.reshape(1, _H),
        batch2d, Wlin, blin.reshape(1, _C),
    )


# speedup vs baseline: 1.0206x; 1.0090x over previous
import functools

import jax
import jax.numpy as jnp
from jax import lax
from jax.experimental import pallas as pl
from jax.experimental.pallas import tpu as pltpu
from jax.experimental.pallas import tpu_sc as plsc

_N = 10000
_E = 320000
_H = 128
_G = 64
_C = 10

_NC = 2
_NS = 16
_NW = _NC * _NS
_EPT = _E // _NW
_EBATCH = 125
_NCH = _EPT // _EBATCH
_HCH = _NCH // 2
_ZR = _N // _NS
_FR = 632
_FL = _N - (_NS - 1) * _FR

_mesh = plsc.VectorSubcoreMesh(
    core_axis_name="c", subcore_axis_name="s", num_cores=_NC, num_subcores=_NS
)


@functools.partial(
    pl.kernel,
    out_type=jax.ShapeDtypeStruct((_NC, _N, _H), jnp.float32),
    mesh=_mesh,
    scratch_types=[
        pltpu.VMEM((2 * _HCH, _EBATCH), jnp.int32),
        pltpu.VMEM((2, _EBATCH, _H), jnp.float32),
        pltpu.VMEM_SHARED((_N, _H), jnp.float32),
        pltpu.SemaphoreType.DMA((2,)),
        pltpu.SemaphoreType.DMA((2,)),
    ],
)
def _seg_sum(h, src, dst, zeros, out, idx_v, rows_v, acc, gsem, ssem):
    c = lax.axis_index("c")
    s = lax.axis_index("s")
    wid = c * _NS + s
    pltpu.sync_copy(src.at[0].at[wid], idx_v.at[pl.ds(0, _HCH)])
    pltpu.sync_copy(dst.at[0].at[wid], idx_v.at[pl.ds(_HCH, _HCH)])
    pltpu.async_copy(h.at[idx_v.at[0]], rows_v.at[0], gsem.at[0])
    pltpu.sync_copy(zeros, acc.at[pl.ds(s * _ZR, _ZR)])
    plsc.subcore_barrier()

    for half in range(2):
        if half == 1:
            pltpu.sync_copy(src.at[1].at[wid], idx_v.at[pl.ds(0, _HCH)])
            pltpu.sync_copy(dst.at[1].at[wid], idx_v.at[pl.ds(_HCH, _HCH)])
            pltpu.async_copy(h.at[idx_v.at[0]], rows_v.at[0], gsem.at[0])

        def body(j, carry):
            nxt = j + 1
            jb = lax.rem(j, 2)

            @pl.when(nxt < _HCH)
            def _():
                nb = lax.rem(nxt, 2)

                @pl.when(nxt >= 2)
                def _():
                    pltpu.make_async_copy(
                        rows_v.at[nb],
                        acc.at[idx_v.at[_HCH + nxt - 2]],
                        ssem.at[nb],
                    ).wait()

                pltpu.async_copy(h.at[idx_v.at[nxt]], rows_v.at[nb], gsem.at[nb])

            pltpu.make_async_copy(
                h.at[idx_v.at[j]], rows_v.at[jb], gsem.at[jb]
            ).wait()
            pltpu.async_copy(
                rows_v.at[jb], acc.at[idx_v.at[_HCH + j]], ssem.at[jb], add=True
            )
            return carry

        lax.fori_loop(0, _HCH, body, 0)
        for jj in (_HCH - 2, _HCH - 1):
            pltpu.make_async_copy(
                rows_v.at[jj % 2], acc.at[idx_v.at[_HCH + jj]], ssem.at[jj % 2]
            ).wait()
    plsc.subcore_barrier()

    @pl.when(s < _NS - 1)
    def _():
        pltpu.sync_copy(
            acc.at[pl.ds(s * _FR, _FR)], out.at[c].at[pl.ds(s * _FR, _FR)]
        )

    @pl.when(s == _NS - 1)
    def _():
        pltpu.sync_copy(
            acc.at[pl.ds((_NS - 1) * _FR, _FL)],
            out.at[c].at[pl.ds((_NS - 1) * _FR, _FL)],
        )


_BR = 1000


def _lin_body(a0, a1, x, wr, ws, b, o):
    y = jnp.dot(a0[...] + a1[...], wr[...], preferred_element_type=jnp.float32)
    y = y + jnp.dot(x[...], ws[...], preferred_element_type=jnp.float32)
    o[...] = jnp.maximum(y + b[...], 0.0)


def _fused_linear_relu(a0, a1, x, wr, ws, b):
    return pl.pallas_call(
        _lin_body,
        grid=(_N // _BR,),
        in_specs=[
            pl.BlockSpec((_BR, _H), lambda i: (i, 0)),
            pl.BlockSpec((_BR, _H), lambda i: (i, 0)),
            pl.BlockSpec((_BR, _H), lambda i: (i, 0)),
            pl.BlockSpec((_H, _H), lambda i: (0, 0)),
            pl.BlockSpec((_H, _H), lambda i: (0, 0)),
            pl.BlockSpec((1, _H), lambda i: (0, 0)),
        ],
        out_specs=pl.BlockSpec((_BR, _H), lambda i: (i, 0)),
        out_shape=jax.ShapeDtypeStruct((_N, _H), jnp.float32),
    )(a0, a1, x, wr, ws, b)


def _last_body(a0, a1, x, wr, ws, b, batch, wlin, blin, o, sums, counts):
    i = pl.program_id(0)

    @pl.when(i == 0)
    def _():
        sums[...] = jnp.zeros_like(sums)
        counts[...] = jnp.zeros_like(counts)

    y = jnp.dot(a0[...] + a1[...], wr[...], preferred_element_type=jnp.float32)
    y = y + jnp.dot(x[...], ws[...], preferred_element_type=jnp.float32)
    y = y + b[...]
    oh = (lax.broadcasted_iota(jnp.int32, (_BR, _G), 1) == batch[...]).astype(
        jnp.float32
    )
    sums[...] += lax.dot_general(
        oh, y, (((0,), (0,)), ((), ())), preferred_element_type=jnp.float32
    )
    counts[...] += lax.dot_general(
        oh,
        jnp.ones((_BR, 1), jnp.float32),
        (((0,), (0,)), ((), ())),
        preferred_element_type=jnp.float32,
    )

    @pl.when(i == pl.num_programs(0) - 1)
    def _():
        pooled = sums[...] / jnp.maximum(counts[...], 1.0)
        o[...] = (
            jnp.dot(pooled, wlin[...], preferred_element_type=jnp.float32) + blin[...]
        )


def _last_layer_pool(a0, a1, x, wr, ws, b, batch2d, wlin, blin):
    return pl.pallas_call(
        _last_body,
        grid=(_N // _BR,),
        in_specs=[
            pl.BlockSpec((_BR, _H), lambda i: (i, 0)),
            pl.BlockSpec((_BR, _H), lambda i: (i, 0)),
            pl.BlockSpec((_BR, _H), lambda i: (i, 0)),
            pl.BlockSpec((_H, _H), lambda i: (0, 0)),
            pl.BlockSpec((_H, _H), lambda i: (0, 0)),
            pl.BlockSpec((1, _H), lambda i: (0, 0)),
            pl.BlockSpec((_BR, 1), lambda i: (i, 0)),
            pl.BlockSpec((_H, _C), lambda i: (0, 0)),
            pl.BlockSpec((1, _C), lambda i: (0, 0)),
        ],
        out_specs=pl.BlockSpec((_G, _C), lambda i: (0, 0)),
        out_shape=jax.ShapeDtypeStruct((_G, _C), jnp.float32),
        scratch_shapes=[
            pltpu.VMEM((_G, _H), jnp.float32),
            pltpu.VMEM((_G, 1), jnp.float32),
        ],
    )(a0, a1, x, wr, ws, b, batch2d, wlin, blin)


def kernel(x, edge_index, batch, W1r, W1s, b1, W2r, W2s, b2, W3r, W3s, b3, Wlin, blin):
    src4 = edge_index[0].reshape(2, _NW, _HCH, _EBATCH)
    dst4 = edge_index[1].reshape(2, _NW, _HCH, _EBATCH)
    zeros = jnp.zeros((_ZR, _H), jnp.float32)
    batch2d = batch.reshape(_N, 1).astype(jnp.int32)

    h = x
    for wr, ws, b in ((W1r, W1s, b1), (W2r, W2s, b2)):
        accs = _seg_sum(h, src4, dst4, zeros)
        h = _fused_linear_relu(accs[0], accs[1], h, wr, ws, b.reshape(1, _H))
    accs = _seg_sum(h, src4, dst4, zeros)
    return _last_layer_pool(
        accs[0], accs[1], h, W3r, W3s, b3.reshape(1, _H),
        batch2d, Wlin, blin.reshape(1, _C),
    )


# speedup vs baseline: 1.0606x; 1.0392x over previous
import functools

import jax
import jax.numpy as jnp
from jax import lax
from jax.experimental import pallas as pl
from jax.experimental.pallas import tpu as pltpu
from jax.experimental.pallas import tpu_sc as plsc

_N = 10000
_E = 320000
_H = 128
_G = 64
_C = 10

_NC = 2
_NS = 16
_NW = _NC * _NS
_EPT = _E // _NW
_EBATCH = 125
_NCH = _EPT // _EBATCH
_HCH = _NCH // 2
_ZR = _N // _NS
_FR = 632
_FL = _N - (_NS - 1) * _FR

_mesh = plsc.VectorSubcoreMesh(
    core_axis_name="c", subcore_axis_name="s", num_cores=_NC, num_subcores=_NS
)


@functools.partial(
    pl.kernel,
    out_type=jax.ShapeDtypeStruct((_NC, _N, _H), jnp.float32),
    mesh=_mesh,
    scratch_types=[
        pltpu.VMEM((2 * _HCH, _EBATCH), jnp.int32),
        pltpu.VMEM((2, _EBATCH, _H), jnp.float32),
        pltpu.VMEM_SHARED((_N, _H), jnp.float32),
        pltpu.SemaphoreType.DMA((2,)),
        pltpu.SemaphoreType.DMA((2,)),
    ],
)
def _seg_sum(h, src, dst, zeros, out, idx_v, rows_v, acc, gsem, ssem):
    c = lax.axis_index("c")
    s = lax.axis_index("s")
    wid = c * _NS + s
    pltpu.sync_copy(src.at[0].at[wid], idx_v.at[pl.ds(0, _HCH)])
    pltpu.sync_copy(dst.at[0].at[wid], idx_v.at[pl.ds(_HCH, _HCH)])
    pltpu.async_copy(h.at[idx_v.at[0]], rows_v.at[0], gsem.at[0])
    pltpu.sync_copy(zeros, acc.at[pl.ds(s * _ZR, _ZR)])
    plsc.subcore_barrier()

    for half in range(2):
        if half == 1:
            pltpu.sync_copy(src.at[1].at[wid], idx_v.at[pl.ds(0, _HCH)])
            pltpu.sync_copy(dst.at[1].at[wid], idx_v.at[pl.ds(_HCH, _HCH)])
            pltpu.async_copy(h.at[idx_v.at[0]], rows_v.at[0], gsem.at[0])

        def body(j, carry):
            nxt = j + 1
            jb = lax.rem(j, 2)

            @pl.when(nxt < _HCH)
            def _():
                nb = lax.rem(nxt, 2)

                @pl.when(nxt >= 2)
                def _():
                    pltpu.make_async_copy(
                        rows_v.at[nb],
                        acc.at[idx_v.at[_HCH + nxt - 2]],
                        ssem.at[nb],
                    ).wait()

                pltpu.async_copy(h.at[idx_v.at[nxt]], rows_v.at[nb], gsem.at[nb])

            pltpu.make_async_copy(
                h.at[idx_v.at[j]], rows_v.at[jb], gsem.at[jb]
            ).wait()
            pltpu.async_copy(
                rows_v.at[jb], acc.at[idx_v.at[_HCH + j]], ssem.at[jb], add=True
            )
            return carry

        lax.fori_loop(0, _HCH, body, 0)
        for jj in (_HCH - 2, _HCH - 1):
            pltpu.make_async_copy(
                rows_v.at[jj % 2], acc.at[idx_v.at[_HCH + jj]], ssem.at[jj % 2]
            ).wait()
    plsc.subcore_barrier()

    @pl.when(s < _NS - 1)
    def _():
        pltpu.sync_copy(
            acc.at[pl.ds(s * _FR, _FR)], out.at[c].at[pl.ds(s * _FR, _FR)]
        )

    @pl.when(s == _NS - 1)
    def _():
        pltpu.sync_copy(
            acc.at[pl.ds((_NS - 1) * _FR, _FL)],
            out.at[c].at[pl.ds((_NS - 1) * _FR, _FL)],
        )


_BR = 1000


def _lin_body(a0, a1, x, wr, ws, b, o):
    y = jnp.dot(a0[0] + a1[0], wr[...], preferred_element_type=jnp.float32)
    y = y + jnp.dot(x[...], ws[...], preferred_element_type=jnp.float32)
    o[...] = jnp.maximum(y + b[...], 0.0)


def _fused_linear_relu(accs, x, wr, ws, b):
    return pl.pallas_call(
        _lin_body,
        grid=(_N // _BR,),
        in_specs=[
            pl.BlockSpec((1, _BR, _H), lambda i: (0, i, 0)),
            pl.BlockSpec((1, _BR, _H), lambda i: (1, i, 0)),
            pl.BlockSpec((_BR, _H), lambda i: (i, 0)),
            pl.BlockSpec((_H, _H), lambda i: (0, 0)),
            pl.BlockSpec((_H, _H), lambda i: (0, 0)),
            pl.BlockSpec((1, _H), lambda i: (0, 0)),
        ],
        out_specs=pl.BlockSpec((_BR, _H), lambda i: (i, 0)),
        out_shape=jax.ShapeDtypeStruct((_N, _H), jnp.float32),
    )(accs, accs, x, wr, ws, b)


def _last_body(a0, a1, x, wr, ws, b, batch, wlin, blin, o, sums, counts):
    i = pl.program_id(0)

    @pl.when(i == 0)
    def _():
        sums[...] = jnp.zeros_like(sums)
        counts[...] = jnp.zeros_like(counts)

    y = jnp.dot(a0[0] + a1[0], wr[...], preferred_element_type=jnp.float32)
    y = y + jnp.dot(x[...], ws[...], preferred_element_type=jnp.float32)
    y = y + b[...]
    oh = (lax.broadcasted_iota(jnp.int32, (_BR, _G), 1) == batch[...]).astype(
        jnp.float32
    )
    sums[...] += lax.dot_general(
        oh, y, (((0,), (0,)), ((), ())), preferred_element_type=jnp.float32
    )
    counts[...] += lax.dot_general(
        oh,
        jnp.ones((_BR, 1), jnp.float32),
        (((0,), (0,)), ((), ())),
        preferred_element_type=jnp.float32,
    )

    @pl.when(i == pl.num_programs(0) - 1)
    def _():
        pooled = sums[...] / jnp.maximum(counts[...], 1.0)
        o[...] = (
            jnp.dot(pooled, wlin[...], preferred_element_type=jnp.float32) + blin[...]
        )


def _last_layer_pool(accs, x, wr, ws, b, batch2d, wlin, blin):
    return pl.pallas_call(
        _last_body,
        grid=(_N // _BR,),
        in_specs=[
            pl.BlockSpec((1, _BR, _H), lambda i: (0, i, 0)),
            pl.BlockSpec((1, _BR, _H), lambda i: (1, i, 0)),
            pl.BlockSpec((_BR, _H), lambda i: (i, 0)),
            pl.BlockSpec((_H, _H), lambda i: (0, 0)),
            pl.BlockSpec((_H, _H), lambda i: (0, 0)),
            pl.BlockSpec((1, _H), lambda i: (0, 0)),
            pl.BlockSpec((_BR, 1), lambda i: (i, 0)),
            pl.BlockSpec((_H, _C), lambda i: (0, 0)),
            pl.BlockSpec((1, _C), lambda i: (0, 0)),
        ],
        out_specs=pl.BlockSpec((_G, _C), lambda i: (0, 0)),
        out_shape=jax.ShapeDtypeStruct((_G, _C), jnp.float32),
        scratch_shapes=[
            pltpu.VMEM((_G, _H), jnp.float32),
            pltpu.VMEM((_G, 1), jnp.float32),
        ],
    )(accs, accs, x, wr, ws, b, batch2d, wlin, blin)


def kernel(x, edge_index, batch, W1r, W1s, b1, W2r, W2s, b2, W3r, W3s, b3, Wlin, blin):
    src4 = edge_index[0].reshape(2, _NW, _HCH, _EBATCH)
    dst4 = edge_index[1].reshape(2, _NW, _HCH, _EBATCH)
    zeros = jnp.zeros((_ZR, _H), jnp.float32)
    batch2d = batch.reshape(_N, 1).astype(jnp.int32)

    h = x
    for wr, ws, b in ((W1r, W1s, b1), (W2r, W2s, b2)):
        accs = _seg_sum(h, src4, dst4, zeros)
        h = _fused_linear_relu(accs, h, wr, ws, b.reshape(1, _H))
    accs = _seg_sum(h, src4, dst4, zeros)
    return _last_layer_pool(
        accs, h, W3r, W3s, b3.reshape(1, _H), batch2d, Wlin, blin.reshape(1, _C)
    )


# speedup vs baseline: 1.0832x; 1.0213x over previous
import functools

import jax
import jax.numpy as jnp
from jax import lax
from jax.experimental import pallas as pl
from jax.experimental.pallas import tpu as pltpu
from jax.experimental.pallas import tpu_sc as plsc

_N = 10000
_E = 320000
_H = 128
_G = 64
_C = 10

_NC = 2
_NS = 16
_NW = _NC * _NS
_EPT = _E // _NW
_EBATCH = 125
_NCH = _EPT // _EBATCH
_HCH = _NCH // 2
_ZR = _N // _NS
_FR = 632
_FL = _N - (_NS - 1) * _FR

_mesh = plsc.VectorSubcoreMesh(
    core_axis_name="c", subcore_axis_name="s", num_cores=_NC, num_subcores=_NS
)


@functools.partial(
    pl.kernel,
    out_type=jax.ShapeDtypeStruct((_NC, _N, _H), jnp.float32),
    mesh=_mesh,
    scratch_types=[
        pltpu.VMEM((2 * _HCH, _EBATCH), jnp.int32),
        pltpu.VMEM((2, _EBATCH, _H), jnp.float32),
        pltpu.VMEM_SHARED((_N, _H), jnp.float32),
        pltpu.SemaphoreType.DMA((2,)),
        pltpu.SemaphoreType.DMA((2,)),
    ],
)
def _seg_sum(h, src, dst, zeros, out, idx_v, rows_v, acc, gsem, ssem):
    c = lax.axis_index("c")
    s = lax.axis_index("s")
    wid = c * _NS + s
    pltpu.sync_copy(src.at[0].at[wid], idx_v.at[pl.ds(0, _HCH)])
    pltpu.sync_copy(dst.at[0].at[wid], idx_v.at[pl.ds(_HCH, _HCH)])
    pltpu.async_copy(h.at[idx_v.at[0]], rows_v.at[0], gsem.at[0])
    pltpu.sync_copy(zeros, acc.at[pl.ds(s * _ZR, _ZR)])
    plsc.subcore_barrier()

    for half in range(2):
        if half == 1:
            pltpu.sync_copy(src.at[1].at[wid], idx_v.at[pl.ds(0, _HCH)])
            pltpu.sync_copy(dst.at[1].at[wid], idx_v.at[pl.ds(_HCH, _HCH)])
            pltpu.async_copy(h.at[idx_v.at[0]], rows_v.at[0], gsem.at[0])

        def body(j, carry):
            nxt = j + 1
            jb = lax.rem(j, 2)

            @pl.when(nxt < _HCH)
            def _():
                nb = lax.rem(nxt, 2)

                @pl.when(nxt >= 2)
                def _():
                    pltpu.make_async_copy(
                        rows_v.at[nb],
                        acc.at[idx_v.at[_HCH + nxt - 2]],
                        ssem.at[nb],
                    ).wait()

                pltpu.async_copy(h.at[idx_v.at[nxt]], rows_v.at[nb], gsem.at[nb])

            pltpu.make_async_copy(
                h.at[idx_v.at[j]], rows_v.at[jb], gsem.at[jb]
            ).wait()
            pltpu.async_copy(
                rows_v.at[jb], acc.at[idx_v.at[_HCH + j]], ssem.at[jb], add=True
            )
            return carry

        lax.fori_loop(0, _HCH, body, 0)
        for jj in (_HCH - 2, _HCH - 1):
            pltpu.make_async_copy(
                rows_v.at[jj % 2], acc.at[idx_v.at[_HCH + jj]], ssem.at[jj % 2]
            ).wait()
    plsc.subcore_barrier()

    @pl.when(s < _NS - 1)
    def _():
        pltpu.sync_copy(
            acc.at[pl.ds(s * _FR, _FR)], out.at[c].at[pl.ds(s * _FR, _FR)]
        )

    @pl.when(s == _NS - 1)
    def _():
        pltpu.sync_copy(
            acc.at[pl.ds((_NS - 1) * _FR, _FL)],
            out.at[c].at[pl.ds((_NS - 1) * _FR, _FL)],
        )


_BR = 2000


def _lin_body(a0, a1, x, wr, ws, b, o):
    y = jnp.dot(a0[0] + a1[0], wr[...], preferred_element_type=jnp.float32)
    y = y + jnp.dot(x[...], ws[...], preferred_element_type=jnp.float32)
    o[...] = jnp.maximum(y + b[...], 0.0)


def _fused_linear_relu(accs, x, wr, ws, b):
    return pl.pallas_call(
        _lin_body,
        grid=(_N // _BR,),
        in_specs=[
            pl.BlockSpec((1, _BR, _H), lambda i: (0, i, 0)),
            pl.BlockSpec((1, _BR, _H), lambda i: (1, i, 0)),
            pl.BlockSpec((_BR, _H), lambda i: (i, 0)),
            pl.BlockSpec((_H, _H), lambda i: (0, 0)),
            pl.BlockSpec((_H, _H), lambda i: (0, 0)),
            pl.BlockSpec((1, _H), lambda i: (0, 0)),
        ],
        out_specs=pl.BlockSpec((_BR, _H), lambda i: (i, 0)),
        out_shape=jax.ShapeDtypeStruct((_N, _H), jnp.float32),
    )(accs, accs, x, wr, ws, b)


def _last_body(a0, a1, x, wr, ws, b, batch, wlin, blin, o, sums, counts):
    i = pl.program_id(0)

    @pl.when(i == 0)
    def _():
        sums[...] = jnp.zeros_like(sums)
        counts[...] = jnp.zeros_like(counts)

    y = jnp.dot(a0[0] + a1[0], wr[...], preferred_element_type=jnp.float32)
    y = y + jnp.dot(x[...], ws[...], preferred_element_type=jnp.float32)
    y = y + b[...]
    oh = (lax.broadcasted_iota(jnp.int32, (_BR, _G), 1) == batch[...]).astype(
        jnp.float32
    )
    sums[...] += lax.dot_general(
        oh, y, (((0,), (0,)), ((), ())), preferred_element_type=jnp.float32
    )
    counts[...] += lax.dot_general(
        oh,
        jnp.ones((_BR, 1), jnp.float32),
        (((0,), (0,)), ((), ())),
        preferred_element_type=jnp.float32,
    )

    @pl.when(i == pl.num_programs(0) - 1)
    def _():
        pooled = sums[...] / jnp.maximum(counts[...], 1.0)
        o[...] = (
            jnp.dot(pooled, wlin[...], preferred_element_type=jnp.float32) + blin[...]
        )


def _last_layer_pool(accs, x, wr, ws, b, batch2d, wlin, blin):
    return pl.pallas_call(
        _last_body,
        grid=(_N // _BR,),
        in_specs=[
            pl.BlockSpec((1, _BR, _H), lambda i: (0, i, 0)),
            pl.BlockSpec((1, _BR, _H), lambda i: (1, i, 0)),
            pl.BlockSpec((_BR, _H), lambda i: (i, 0)),
            pl.BlockSpec((_H, _H), lambda i: (0, 0)),
            pl.BlockSpec((_H, _H), lambda i: (0, 0)),
            pl.BlockSpec((1, _H), lambda i: (0, 0)),
            pl.BlockSpec((_BR, 1), lambda i: (i, 0)),
            pl.BlockSpec((_H, _C), lambda i: (0, 0)),
            pl.BlockSpec((1, _C), lambda i: (0, 0)),
        ],
        out_specs=pl.BlockSpec((_G, _C), lambda i: (0, 0)),
        out_shape=jax.ShapeDtypeStruct((_G, _C), jnp.float32),
        scratch_shapes=[
            pltpu.VMEM((_G, _H), jnp.float32),
            pltpu.VMEM((_G, 1), jnp.float32),
        ],
    )(accs, accs, x, wr, ws, b, batch2d, wlin, blin)


def kernel(x, edge_index, batch, W1r, W1s, b1, W2r, W2s, b2, W3r, W3s, b3, Wlin, blin):
    src4 = edge_index[0].reshape(2, _NW, _HCH, _EBATCH)
    dst4 = edge_index[1].reshape(2, _NW, _HCH, _EBATCH)
    zeros = jnp.zeros((_ZR, _H), jnp.float32)
    batch2d = batch.reshape(_N, 1).astype(jnp.int32)

    h = x
    for wr, ws, b in ((W1r, W1s, b1), (W2r, W2s, b2)):
        accs = _seg_sum(h, src4, dst4, zeros)
        h = _fused_linear_relu(accs, h, wr, ws, b.reshape(1, _H))
    accs = _seg_sum(h, src4, dst4, zeros)
    return _last_layer_pool(
        accs, h, W3r, W3s, b3.reshape(1, _H), batch2d, Wlin, blin.reshape(1, _C)
    )


# speedup vs baseline: 1.0909x; 1.0071x over previous
import functools

import jax
import jax.numpy as jnp
from jax import lax
from jax.experimental import pallas as pl
from jax.experimental.pallas import tpu as pltpu
from jax.experimental.pallas import tpu_sc as plsc

_N = 10000
_E = 320000
_H = 128
_G = 64
_C = 10

_NC = 2
_NS = 16
_NW = _NC * _NS
_EPT = _E // _NW
_EBATCH = 125
_NCH = _EPT // _EBATCH
_HCH = _NCH // 2
_ZR = _N // _NS
_FR = 632
_FL = _N - (_NS - 1) * _FR

_mesh = plsc.VectorSubcoreMesh(
    core_axis_name="c", subcore_axis_name="s", num_cores=_NC, num_subcores=_NS
)


@functools.partial(
    pl.kernel,
    out_type=jax.ShapeDtypeStruct((_NC, _N, _H), jnp.float32),
    mesh=_mesh,
    scratch_types=[
        pltpu.VMEM((2 * _HCH, _EBATCH), jnp.int32),
        pltpu.VMEM((2, _EBATCH, _H), jnp.float32),
        pltpu.VMEM_SHARED((_N, _H), jnp.float32),
        pltpu.SemaphoreType.DMA((2,)),
        pltpu.SemaphoreType.DMA((2,)),
    ],
)
def _seg_sum(h, src, dst, zeros, out, idx_v, rows_v, acc, gsem, ssem):
    c = lax.axis_index("c")
    s = lax.axis_index("s")
    wid = c * _NS + s
    pltpu.sync_copy(src.at[0].at[wid], idx_v.at[pl.ds(0, _HCH)])
    pltpu.sync_copy(dst.at[0].at[wid], idx_v.at[pl.ds(_HCH, _HCH)])
    pltpu.async_copy(h.at[idx_v.at[0]], rows_v.at[0], gsem.at[0])
    pltpu.sync_copy(zeros, acc.at[pl.ds(s * _ZR, _ZR)])
    plsc.subcore_barrier()

    for half in range(2):
        if half == 1:
            pltpu.sync_copy(src.at[1].at[wid], idx_v.at[pl.ds(0, _HCH)])
            pltpu.sync_copy(dst.at[1].at[wid], idx_v.at[pl.ds(_HCH, _HCH)])
            pltpu.async_copy(h.at[idx_v.at[0]], rows_v.at[0], gsem.at[0])

        def body(j, carry):
            nxt = j + 1
            jb = lax.rem(j, 2)

            @pl.when(nxt < _HCH)
            def _():
                nb = lax.rem(nxt, 2)

                @pl.when(nxt >= 2)
                def _():
                    pltpu.make_async_copy(
                        rows_v.at[nb],
                        acc.at[idx_v.at[_HCH + nxt - 2]],
                        ssem.at[nb],
                    ).wait()

                pltpu.async_copy(h.at[idx_v.at[nxt]], rows_v.at[nb], gsem.at[nb])

            pltpu.make_async_copy(
                h.at[idx_v.at[j]], rows_v.at[jb], gsem.at[jb]
            ).wait()
            pltpu.async_copy(
                rows_v.at[jb], acc.at[idx_v.at[_HCH + j]], ssem.at[jb], add=True
            )
            return carry

        lax.fori_loop(0, _HCH, body, 0)
        for jj in (_HCH - 2, _HCH - 1):
            pltpu.make_async_copy(
                rows_v.at[jj % 2], acc.at[idx_v.at[_HCH + jj]], ssem.at[jj % 2]
            ).wait()
    plsc.subcore_barrier()

    @pl.when(s < _NS - 1)
    def _():
        pltpu.sync_copy(
            acc.at[pl.ds(s * _FR, _FR)], out.at[c].at[pl.ds(s * _FR, _FR)]
        )

    @pl.when(s == _NS - 1)
    def _():
        pltpu.sync_copy(
            acc.at[pl.ds((_NS - 1) * _FR, _FL)],
            out.at[c].at[pl.ds((_NS - 1) * _FR, _FL)],
        )


_BR = 5000


def _lin_body(a0, a1, x, wr, ws, b, o):
    y = jnp.dot(a0[0] + a1[0], wr[...], preferred_element_type=jnp.float32)
    y = y + jnp.dot(x[...], ws[...], preferred_element_type=jnp.float32)
    o[...] = jnp.maximum(y + b[...], 0.0)


def _fused_linear_relu(accs, x, wr, ws, b):
    return pl.pallas_call(
        _lin_body,
        grid=(_N // _BR,),
        in_specs=[
            pl.BlockSpec((1, _BR, _H), lambda i: (0, i, 0)),
            pl.BlockSpec((1, _BR, _H), lambda i: (1, i, 0)),
            pl.BlockSpec((_BR, _H), lambda i: (i, 0)),
            pl.BlockSpec((_H, _H), lambda i: (0, 0)),
            pl.BlockSpec((_H, _H), lambda i: (0, 0)),
            pl.BlockSpec((1, _H), lambda i: (0, 0)),
        ],
        out_specs=pl.BlockSpec((_BR, _H), lambda i: (i, 0)),
        out_shape=jax.ShapeDtypeStruct((_N, _H), jnp.float32),
    )(accs, accs, x, wr, ws, b)


def _last_body(a0, a1, x, wr, ws, b, batch, wlin, blin, o, sums, counts):
    i = pl.program_id(0)

    @pl.when(i == 0)
    def _():
        sums[...] = jnp.zeros_like(sums)
        counts[...] = jnp.zeros_like(counts)

    y = jnp.dot(a0[0] + a1[0], wr[...], preferred_element_type=jnp.float32)
    y = y + jnp.dot(x[...], ws[...], preferred_element_type=jnp.float32)
    y = y + b[...]
    oh = (lax.broadcasted_iota(jnp.int32, (_BR, _G), 1) == batch[...]).astype(
        jnp.float32
    )
    sums[...] += lax.dot_general(
        oh, y, (((0,), (0,)), ((), ())), preferred_element_type=jnp.float32
    )
    counts[...] += lax.dot_general(
        oh,
        jnp.ones((_BR, 1), jnp.float32),
        (((0,), (0,)), ((), ())),
        preferred_element_type=jnp.float32,
    )

    @pl.when(i == pl.num_programs(0) - 1)
    def _():
        pooled = sums[...] / jnp.maximum(counts[...], 1.0)
        o[...] = (
            jnp.dot(pooled, wlin[...], preferred_element_type=jnp.float32) + blin[...]
        )


def _last_layer_pool(accs, x, wr, ws, b, batch2d, wlin, blin):
    return pl.pallas_call(
        _last_body,
        grid=(_N // _BR,),
        in_specs=[
            pl.BlockSpec((1, _BR, _H), lambda i: (0, i, 0)),
            pl.BlockSpec((1, _BR, _H), lambda i: (1, i, 0)),
            pl.BlockSpec((_BR, _H), lambda i: (i, 0)),
            pl.BlockSpec((_H, _H), lambda i: (0, 0)),
            pl.BlockSpec((_H, _H), lambda i: (0, 0)),
            pl.BlockSpec((1, _H), lambda i: (0, 0)),
            pl.BlockSpec((_BR, 1), lambda i: (i, 0)),
            pl.BlockSpec((_H, _C), lambda i: (0, 0)),
            pl.BlockSpec((1, _C), lambda i: (0, 0)),
        ],
        out_specs=pl.BlockSpec((_G, _C), lambda i: (0, 0)),
        out_shape=jax.ShapeDtypeStruct((_G, _C), jnp.float32),
        scratch_shapes=[
            pltpu.VMEM((_G, _H), jnp.float32),
            pltpu.VMEM((_G, 1), jnp.float32),
        ],
    )(accs, accs, x, wr, ws, b, batch2d, wlin, blin)


def kernel(x, edge_index, batch, W1r, W1s, b1, W2r, W2s, b2, W3r, W3s, b3, Wlin, blin):
    src4 = edge_index[0].reshape(2, _NW, _HCH, _EBATCH)
    dst4 = edge_index[1].reshape(2, _NW, _HCH, _EBATCH)
    zeros = jnp.zeros((_ZR, _H), jnp.float32)
    batch2d = batch.reshape(_N, 1).astype(jnp.int32)

    h = x
    for wr, ws, b in ((W1r, W1s, b1), (W2r, W2s, b2)):
        accs = _seg_sum(h, src4, dst4, zeros)
        h = _fused_linear_relu(accs, h, wr, ws, b.reshape(1, _H))
    accs = _seg_sum(h, src4, dst4, zeros)
    return _last_layer_pool(
        accs, h, W3r, W3s, b3.reshape(1, _H), batch2d, Wlin, blin.reshape(1, _C)
    )
